# baseline (speedup 1.0000x reference)
import functools

import jax
import jax.numpy as jnp
from jax import lax
from jax.experimental import pallas as pl
from jax.experimental.pallas import tpu as pltpu
from jax.experimental.pallas import tpu_sc as plsc

_TV = 4096
_K = 4
_LOG2E = 1.4426950408889634
_LN2 = 0.6931471805599453

_CONTRACT0 = (((0,), (0,)), ((), ()))


@functools.lru_cache(maxsize=None)
def _make_sc_gather(V, D, B):
    info = plsc.get_sparse_core_info()
    NC, NS, L = info.num_cores, info.num_subcores, info.num_lanes
    NW = NC * NS
    assert B % (L * NW) == 0 and D == L and 128 % D == 0
    b_per_w = B // NW
    rows_per_line = 128 // D
    mesh = plsc.VectorSubcoreMesh(core_axis_name="c", subcore_axis_name="s")

    @functools.partial(
        pl.kernel,
        mesh=mesh,
        out_type=jax.ShapeDtypeStruct((B, D), jnp.float32),
        scratch_types=[
            pltpu.VMEM((b_per_w,), jnp.int32),
            pltpu.VMEM((b_per_w,), jnp.int32),
            pltpu.VMEM((b_per_w, 128), jnp.float32),
            pltpu.VMEM((b_per_w, D), jnp.float32),
            pltpu.SemaphoreType.DMA,
        ],
        compiler_params=pltpu.CompilerParams(needs_layout_passes=False),
    )
    def gather_kernel(idx_hbm, table_hbm, out_hbm, idx_v, line_v, rows_v,
                      x_v, sem):
        wid = lax.axis_index("s") * NC + lax.axis_index("c")
        base = wid * b_per_w
        pltpu.sync_copy(idx_hbm.at[pl.ds(base, b_per_w)], idx_v)
        for g in range(b_per_w // L):
            iv = idx_v[pl.ds(g * L, L)]
            line_v[pl.ds(g * L, L)] = lax.div(iv, rows_per_line)
        pltpu.async_copy(table_hbm.at[line_v], rows_v, sem).wait()
        for g in range(b_per_w // L):
            iv = idx_v[pl.ds(g * L, L)]
            off = lax.rem(iv, rows_per_line) * D
            row = lax.iota(jnp.int32, L) + g * L
            for k in range(D):
                vals = plsc.load_gather(rows_v, [row, off + k])
                plsc.store_scatter(
                    x_v, [row, jnp.full((L,), k, jnp.int32)], vals
                )
        pltpu.sync_copy(x_v, out_hbm.at[pl.ds(base, b_per_w)])

    return gather_kernel


def _fused_body(xT_ref, W1aug_ref, W2_ref, b2_ref, out_ref, m_scr, s_scr,
                lse_scr, *, V, nj, K, Bc):
    p = pl.program_id(0)
    j = pl.program_id(1)
    TV = W2_ref.shape[1]
    ones_c = jnp.full((1, Bc), 1.0, dtype=jnp.float32)
    ones_v = jnp.full((1, TV), 1.0, dtype=jnp.float32)

    @pl.when(jnp.logical_and(p == 0, j == 0))
    def _init():
        m_scr[...] = jnp.zeros_like(m_scr)
        s_scr[...] = jnp.zeros_like(s_scr)

    def _haug(cols):
        xaug = jnp.concatenate([xT_ref[:, cols], ones_c], axis=0)
        hT = jnp.maximum(
            lax.dot_general(W1aug_ref[...], xaug, _CONTRACT0,
                            preferred_element_type=jnp.float32),
            0.0,
        )
        return jnp.concatenate([hT, ones_c], axis=0)

    @pl.when(p < K)
    def _stats():
        cols = pl.ds(p * Bc, Bc)
        haug_s = jnp.concatenate(
            [_haug(cols) * _LOG2E, -m_scr[0:1, cols]], axis=0
        )
        mask = j * TV + lax.broadcasted_iota(jnp.int32, (1, TV), 1) < V
        w2m = jnp.where(mask, W2_ref[...], 0.0)
        b2m = jnp.where(mask, b2_ref[...], -jnp.inf)
        w2aug = jnp.concatenate([w2m, b2m, ones_v], axis=0)
        t = lax.dot_general(w2aug, haug_s, _CONTRACT0,
                            preferred_element_type=jnp.float32)

        tmax = jnp.max(t, axis=0, keepdims=True)
        rescale = jnp.logical_or(j == 0, jnp.max(tmax) > 60.0)

        @pl.when(jnp.logical_not(rescale))
        def _fast():
            s_scr[0:1, cols] = s_scr[0:1, cols] + jnp.sum(
                jnp.exp2(t), axis=0, keepdims=True
            )

        @pl.when(rescale)
        def _slow():
            shift = jnp.where(j == 0, tmax, jnp.maximum(tmax, 0.0))
            prev = jnp.where(j == 0, 0.0,
                             s_scr[0:1, cols] * jnp.exp2(-shift))
            s_scr[0:1, cols] = prev + jnp.sum(
                jnp.exp2(t - shift), axis=0, keepdims=True
            )
            m_scr[0:1, cols] = m_scr[0:1, cols] + shift

        @pl.when(j == nj - 1)
        def _fin():
            lse_scr[0:1, cols] = (
                m_scr[0:1, cols] + jnp.log2(s_scr[0:1, cols])
            ) * _LN2

    @pl.when(p > 0)
    def _write():
        cols = pl.ds((p - 1) * Bc, Bc)
        haug_w = jnp.concatenate(
            [_haug(cols), -lse_scr[0:1, cols]], axis=0
        )
        w2aug = jnp.concatenate([W2_ref[...], b2_ref[...], ones_v], axis=0)
        out_ref[...] = lax.dot_general(w2aug, haug_w, _CONTRACT0,
                                       preferred_element_type=jnp.float32)


def _mlp_logsoftmax(xT, W1, b1, W2, b2):
    E, B = xT.shape
    H, V = W2.shape
    nj = pl.cdiv(V, _TV)
    Bc = B // _K
    W1aug = jnp.concatenate([W1, b1.reshape(1, H)], axis=0)
    b2r = b2.reshape(1, V)

    full = lambda shape: pl.BlockSpec(shape, lambda p, j: (0, 0))
    out_t = pl.pallas_call(
        functools.partial(_fused_body, V=V, nj=nj, K=_K, Bc=Bc),
        grid=(_K + 1, nj),
        in_specs=[
            full((E, B)),
            full((E + 1, H)),
            pl.BlockSpec((H, _TV), lambda p, j: (0, j)),
            pl.BlockSpec((1, _TV), lambda p, j: (0, j)),
        ],
        out_specs=pl.BlockSpec(
            (_TV, Bc),
            lambda p, j: (jnp.where(p == 0, 0, j), jnp.maximum(p - 1, 0)),
        ),
        out_shape=jax.ShapeDtypeStruct((V, B), jnp.float32),
        scratch_shapes=[
            pltpu.VMEM((8, B), jnp.float32),
            pltpu.VMEM((8, B), jnp.float32),
            pltpu.VMEM((8, B), jnp.float32),
        ],
        compiler_params=pltpu.CompilerParams(
            dimension_semantics=("arbitrary", "arbitrary")
        ),
    )(xT, W1aug, W2, b2r)
    return out_t.T


def kernel(inputs, emb, W1, b1, W2, b2):
    V, E = emb.shape
    (B,) = inputs.shape
    table = emb.reshape(V * E // 128, 128)
    x = _make_sc_gather(V, E, B)(inputs.astype(jnp.int32), table)
    return _mlp_logsoftmax(x.T, W1, b1, W2, b2)


# speedup vs baseline: 1.1372x; 1.1372x over previous
import functools

import jax
import jax.numpy as jnp
from jax import lax
from jax.experimental import pallas as pl
from jax.experimental.pallas import tpu as pltpu
from jax.experimental.pallas import tpu_sc as plsc

_TV = 4096
_K = 2
_LOG2E = 1.4426950408889634
_LN2 = 0.6931471805599453

_CONTRACT0 = (((0,), (0,)), ((), ()))


@functools.lru_cache(maxsize=None)
def _make_sc_gather(V, D, B):
    info = plsc.get_sparse_core_info()
    NC, NS, L = info.num_cores, info.num_subcores, info.num_lanes
    NW = NC * NS
    assert B % (L * NW) == 0 and D == L and 128 % D == 0
    b_per_w = B // NW
    rows_per_line = 128 // D
    mesh = plsc.VectorSubcoreMesh(core_axis_name="c", subcore_axis_name="s")

    @functools.partial(
        pl.kernel,
        mesh=mesh,
        out_type=jax.ShapeDtypeStruct((B, D), jnp.float32),
        scratch_types=[
            pltpu.VMEM((b_per_w,), jnp.int32),
            pltpu.VMEM((b_per_w,), jnp.int32),
            pltpu.VMEM((b_per_w, 128), jnp.float32),
            pltpu.VMEM((b_per_w, D), jnp.float32),
            pltpu.SemaphoreType.DMA,
        ],
        compiler_params=pltpu.CompilerParams(needs_layout_passes=False),
    )
    def gather_kernel(idx_hbm, table_hbm, out_hbm, idx_v, line_v, rows_v,
                      x_v, sem):
        wid = lax.axis_index("s") * NC + lax.axis_index("c")
        base = wid * b_per_w
        pltpu.sync_copy(idx_hbm.at[pl.ds(base, b_per_w)], idx_v)
        for g in range(b_per_w // L):
            iv = idx_v[pl.ds(g * L, L)]
            line_v[pl.ds(g * L, L)] = lax.div(iv, rows_per_line)
        pltpu.async_copy(table_hbm.at[line_v], rows_v, sem).wait()
        for g in range(b_per_w // L):
            iv = idx_v[pl.ds(g * L, L)]
            off = lax.rem(iv, rows_per_line) * D
            row = lax.iota(jnp.int32, L) + g * L
            for k in range(D):
                vals = plsc.load_gather(rows_v, [row, off + k])
                plsc.store_scatter(
                    x_v, [row, jnp.full((L,), k, jnp.int32)], vals
                )
        pltpu.sync_copy(x_v, out_hbm.at[pl.ds(base, b_per_w)])

    return gather_kernel


def _fused_body(xT_ref, W1aug_ref, W2_ref, b2_ref, out_ref, m_scr, s_scr,
                lse_scr, *, V, nj, K, Bc):
    p = pl.program_id(0)
    j = pl.program_id(1)
    TV = W2_ref.shape[1]
    ones_c = jnp.full((1, Bc), 1.0, dtype=jnp.float32)
    ones_v = jnp.full((1, TV), 1.0, dtype=jnp.float32)

    @pl.when(jnp.logical_and(p == 0, j == 0))
    def _init():
        m_scr[...] = jnp.zeros_like(m_scr)
        s_scr[...] = jnp.zeros_like(s_scr)

    def _haug(cols):
        xaug = jnp.concatenate([xT_ref[:, cols], ones_c], axis=0)
        hT = jnp.maximum(
            lax.dot_general(W1aug_ref[...], xaug, _CONTRACT0,
                            preferred_element_type=jnp.float32),
            0.0,
        )
        return jnp.concatenate([hT, ones_c], axis=0)

    @pl.when(p < K)
    def _stats():
        cols = pl.ds(p * Bc, Bc)
        haug_s = jnp.concatenate(
            [_haug(cols) * _LOG2E, -m_scr[0:1, cols]], axis=0
        )
        mask = j * TV + lax.broadcasted_iota(jnp.int32, (1, TV), 1) < V
        w2m = jnp.where(mask, W2_ref[...], 0.0)
        b2m = jnp.where(mask, b2_ref[...], -jnp.inf)
        w2aug = jnp.concatenate([w2m, b2m, ones_v], axis=0)
        t = lax.dot_general(w2aug, haug_s, _CONTRACT0,
                            preferred_element_type=jnp.float32)

        tmax = jnp.max(t, axis=0, keepdims=True)
        rescale = jnp.logical_or(j == 0, jnp.max(tmax) > 60.0)

        @pl.when(jnp.logical_not(rescale))
        def _fast():
            s_scr[0:1, cols] = s_scr[0:1, cols] + jnp.sum(
                jnp.exp2(t), axis=0, keepdims=True
            )

        @pl.when(rescale)
        def _slow():
            shift = jnp.where(j == 0, tmax, jnp.maximum(tmax, 0.0))
            prev = jnp.where(j == 0, 0.0,
                             s_scr[0:1, cols] * jnp.exp2(-shift))
            s_scr[0:1, cols] = prev + jnp.sum(
                jnp.exp2(t - shift), axis=0, keepdims=True
            )
            m_scr[0:1, cols] = m_scr[0:1, cols] + shift

        @pl.when(j == nj - 1)
        def _fin():
            lse_scr[0:1, cols] = (
                m_scr[0:1, cols] + jnp.log2(s_scr[0:1, cols])
            ) * _LN2

    @pl.when(p > 0)
    def _write():
        cols = pl.ds((p - 1) * Bc, Bc)
        haug_w = jnp.concatenate(
            [_haug(cols), -lse_scr[0:1, cols]], axis=0
        )
        w2aug = jnp.concatenate([W2_ref[...], b2_ref[...], ones_v], axis=0)
        out_ref[...] = lax.dot_general(w2aug, haug_w, _CONTRACT0,
                                       preferred_element_type=jnp.float32)


def _mlp_logsoftmax(xT, W1, b1, W2, b2):
    E, B = xT.shape
    H, V = W2.shape
    nj = pl.cdiv(V, _TV)
    Bc = B // _K
    W1aug = jnp.concatenate([W1, b1.reshape(1, H)], axis=0)
    b2r = b2.reshape(1, V)

    full = lambda shape: pl.BlockSpec(shape, lambda p, j: (0, 0))
    out_t = pl.pallas_call(
        functools.partial(_fused_body, V=V, nj=nj, K=_K, Bc=Bc),
        grid=(_K + 1, nj),
        in_specs=[
            full((E, B)),
            full((E + 1, H)),
            pl.BlockSpec((H, _TV), lambda p, j: (0, j)),
            pl.BlockSpec((1, _TV), lambda p, j: (0, j)),
        ],
        out_specs=pl.BlockSpec(
            (_TV, Bc),
            lambda p, j: (jnp.where(p == 0, 0, j), jnp.maximum(p - 1, 0)),
        ),
        out_shape=jax.ShapeDtypeStruct((V, B), jnp.float32),
        scratch_shapes=[
            pltpu.VMEM((8, B), jnp.float32),
            pltpu.VMEM((8, B), jnp.float32),
            pltpu.VMEM((8, B), jnp.float32),
        ],
        compiler_params=pltpu.CompilerParams(
            dimension_semantics=("arbitrary", "arbitrary")
        ),
    )(xT, W1aug, W2, b2r)
    return out_t.T


def kernel(inputs, emb, W1, b1, W2, b2):
    V, E = emb.shape
    (B,) = inputs.shape
    table = emb.reshape(V * E // 128, 128)
    x = _make_sc_gather(V, E, B)(inputs.astype(jnp.int32), table)
    return _mlp_logsoftmax(x.T, W1, b1, W2, b2)
